# baseline (speedup 1.0000x reference)
import functools

import jax
import jax.numpy as jnp
from jax import lax
from jax.experimental import pallas as pl
from jax.experimental.pallas import tpu as pltpu
from jax.experimental.pallas import tpu_sc as plsc

B = 16
NC, NS = 2, 16
NW = NC * NS
CHUNK = 64
N_CHUNKS = 50
N_PAIRS = N_CHUNKS // 2
TRIS_PER_W = CHUNK * N_CHUNKS
T_PAD = NW * TRIS_PER_W
WORDS = 2 * B
N_VERTS = 50000
VB = 64
N_FULL_BLKS = N_VERTS // VB
TAIL_V = N_VERTS - N_FULL_BLKS * VB
VW = 3 * VB
VROW = 193


def _sc_volume(x, mblk):
    mesh = plsc.VectorSubcoreMesh(core_axis_name="c", subcore_axis_name="s")

    @functools.partial(
        pl.kernel,
        mesh=mesh,
        out_type=jax.ShapeDtypeStruct((NW, B), jnp.float32),
        compiler_params=pltpu.CompilerParams(use_tc_tiling_on_sc=False,
                                             needs_layout_passes=False),
        scratch_types=[
            pltpu.VMEM((CHUNK, 3), jnp.int32),
            pltpu.VMEM((CHUNK, 3), jnp.int32),
            pltpu.VMEM((3, CHUNK), jnp.int32),
            pltpu.VMEM((3, CHUNK), jnp.int32),
            pltpu.VMEM((3, CHUNK, WORDS), jnp.int32),
            pltpu.VMEM((3, CHUNK, WORDS), jnp.int32),
            pltpu.VMEM((B, VROW), jnp.float32),
            pltpu.VMEM((B, VROW), jnp.float32),
            pltpu.VMEM((VB, WORDS), jnp.int32),
            pltpu.VMEM((VB, WORDS), jnp.int32),
            pltpu.VMEM((B,), jnp.float32),
            pltpu.VMEM_SHARED((N_VERTS, WORDS), jnp.int32),
            pltpu.SemaphoreType.DMA,
            pltpu.SemaphoreType.DMA,
            pltpu.SemaphoreType.DMA,
            pltpu.SemaphoreType.DMA,
            pltpu.SemaphoreType.DMA,
            pltpu.SemaphoreType.DMA,
        ],
    )
    def k(x_hbm, m_hbm, out_hbm,
          ma, mb, ia, ib, ra, rb, va, vb_, wa, wb, accv, tab,
          sem_a, sem_b, sem_va, sem_vb, sem_wa, sem_wb):
        cid = lax.axis_index("c")
        sid = lax.axis_index("s")
        wid = sid * NC + cid

        lane = lax.broadcasted_iota(jnp.int32, (B,), 0)
        zero16 = jnp.zeros((B,), jnp.float32)

        def stage(vbuf, sem, g):
            pltpu.async_copy(x_hbm.at[:, pl.ds(g * VW, VW)],
                             vbuf.at[:, pl.ds(0, VW)], sem)

        def stage_wait(vbuf, sem, g):
            pltpu.make_async_copy(x_hbm.at[:, pl.ds(g * VW, VW)],
                                  vbuf.at[:, pl.ds(0, VW)], sem).wait()

        def flush(wbuf, sem, g):
            pltpu.async_copy(wbuf, tab.at[pl.ds(g * VB, VB)], sem)

        def flush_wait(wbuf, sem, g):
            pltpu.make_async_copy(wbuf, tab.at[pl.ds(g * VB, VB)],
                                  sem).wait()

        def build(vbuf, wbuf, nv):
            def vert(dv, _):
                col = 3 * dv
                c0 = plsc.load_gather(vbuf, [lane, jnp.full((B,), col,
                                                            jnp.int32)])
                c1 = plsc.load_gather(vbuf, [lane, jnp.full((B,), col + 1,
                                                            jnp.int32)])
                c2 = plsc.load_gather(vbuf, [lane, jnp.full((B,), col + 2,
                                                            jnp.int32)])
                w01 = plsc.bitcast(
                    plsc.pack(c0, c1, format=plsc.PackFormat.INTERLEAVED),
                    jnp.int32)
                w2 = plsc.bitcast(
                    plsc.pack(c2, zero16,
                              format=plsc.PackFormat.INTERLEAVED),
                    jnp.int32)
                wbuf[dv, pl.ds(0, B)] = w01
                wbuf[dv, pl.ds(B, B)] = w2
                return 0

            lax.fori_loop(0, nv, vert, 0, unroll=4)

        n_sb = jnp.where(sid < N_FULL_BLKS - 48 * NS, 49, 48)
        g0 = sid

        stage(va, sem_va, g0)

        def build_pair(p, _):
            gA = g0 + 32 * p
            gB = gA + 16

            @pl.when(2 * p < n_sb)
            def _():
                @pl.when(2 * p + 1 < n_sb)
                def _():
                    stage(vb_, sem_vb, gB)

                @pl.when(p > 0)
                def _():
                    flush_wait(wa, sem_wa, gA)
                stage_wait(va, sem_va, gA)
                build(va, wa, VB)
                flush(wa, sem_wa, gA)

                @pl.when(2 * p + 2 < n_sb)
                def _():
                    stage(va, sem_va, gA + 32)

            @pl.when(2 * p + 1 < n_sb)
            def _():
                @pl.when(p > 0)
                def _():
                    flush_wait(wb, sem_wb, gB)
                stage_wait(vb_, sem_vb, gB)
                build(vb_, wb, VB)
                flush(wb, sem_wb, gB)

            return 0

        lax.fori_loop(0, 25, build_pair, 0)

        @pl.when(n_sb >= 1)
        def _():
            flush_wait(wa, sem_wa, 0)

        @pl.when(n_sb >= 2)
        def _():
            flush_wait(wb, sem_wb, 0)

        @pl.when(sid == 0)
        def _():
            pltpu.sync_copy(x_hbm.at[:, pl.ds(N_FULL_BLKS * VW, 3 * TAIL_V)],
                            va.at[:, pl.ds(0, 3 * TAIL_V)])
            build(va, wa, TAIL_V)
            pltpu.sync_copy(wa.at[pl.ds(0, TAIL_V)],
                            tab.at[pl.ds(N_FULL_BLKS * VB, TAIL_V)])

        plsc.subcore_barrier()

        def fetch(mbuf, ibuf, rbuf, sem, ci):
            pltpu.sync_copy(m_hbm.at[wid * N_CHUNKS + ci], mbuf)
            for s in range(3):
                scol = jnp.full((B,), s, jnp.int32)
                for g in range(CHUNK // B):
                    v = plsc.load_gather(mbuf, [g * B + lane, scol])
                    ibuf[s, pl.ds(g * B, B)] = v
            for s in range(3):
                pltpu.async_copy(tab.at[ibuf.at[s]], rbuf.at[s], sem)

        def wait(ibuf, rbuf, sem):
            for s in range(3):
                pltpu.make_async_copy(tab.at[ibuf.at[s]],
                                      rbuf.at[s], sem).wait()

        def comp3(rbuf, s, t):
            w01 = rbuf[s, t, pl.ds(0, B)]
            c0, c1 = plsc.unpack(plsc.bitcast(w01, jnp.bfloat16),
                                 format=plsc.PackFormat.INTERLEAVED,
                                 preferred_element_type=jnp.float32)
            w2 = rbuf[s, t, pl.ds(B, B)]
            c2, _ = plsc.unpack(plsc.bitcast(w2, jnp.bfloat16),
                                format=plsc.PackFormat.INTERLEAVED,
                                preferred_element_type=jnp.float32)
            return c0, c1, c2

        def compute(rbuf, acc):
            def tri_body(t, a):
                a1, a2, a3 = comp3(rbuf, 0, t)
                b1, b2, b3 = comp3(rbuf, 1, t)
                d1, d2, d3 = comp3(rbuf, 2, t)
                det = (a1 * (b2 * d3 - b3 * d2)
                       - a2 * (b1 * d3 - b3 * d1)
                       + a3 * (b1 * d2 - b2 * d1))
                return a + jnp.abs(det)

            return lax.fori_loop(0, CHUNK, tri_body, acc, unroll=4)

        fetch(ma, ia, ra, sem_a, 0)

        def pair_body(p, acc):
            c0 = 2 * p
            fetch(mb, ib, rb, sem_b, c0 + 1)
            wait(ia, ra, sem_a)
            acc = compute(ra, acc)

            @pl.when(p < N_PAIRS - 1)
            def _():
                fetch(ma, ia, ra, sem_a, c0 + 2)

            wait(ib, rb, sem_b)
            return compute(rb, acc)

        acc = lax.fori_loop(0, N_PAIRS, pair_body,
                            jnp.zeros((B,), jnp.float32))
        accv[...] = acc
        pltpu.sync_copy(accv, out_hbm.at[wid])

    return k(x, mblk)


_XBLK = 12288


def _normalize(x, partials):
    cols = x.shape[1]
    grid = pl.cdiv(cols, _XBLK)

    def body(p_ref, x_ref, o_ref):
        tot = jnp.sum(p_ref[...], axis=0)
        vol = tot * (1.0 / 6.0)
        inv = jnp.exp(jnp.log(vol) * (-1.0 / 3.0))
        o_ref[...] = x_ref[...] * inv[:, None]

    return pl.pallas_call(
        body,
        grid=(grid,),
        in_specs=[
            pl.BlockSpec((NW, B), lambda i: (0, 0)),
            pl.BlockSpec((B, _XBLK), lambda i: (0, i)),
        ],
        out_specs=pl.BlockSpec((B, _XBLK), lambda i: (0, i)),
        out_shape=jax.ShapeDtypeStruct(x.shape, x.dtype),
    )(partials, x)


def kernel(x, M):
    Mi = M.astype(jnp.int32)
    pad = T_PAD - Mi.shape[0]
    Mp = jnp.concatenate([Mi, jnp.zeros((pad, 3), jnp.int32)], axis=0)
    mblk = Mp.reshape(NW * N_CHUNKS, CHUNK, 3)
    partials = _sc_volume(x, mblk)
    return _normalize(x, partials)


# speedup vs baseline: 2.0451x; 2.0451x over previous
import functools

import jax
import jax.numpy as jnp
from jax import lax
from jax.experimental import pallas as pl
from jax.experimental.pallas import tpu as pltpu
from jax.experimental.pallas import tpu_sc as plsc

B = 16
NC, NS = 2, 16
NW = NC * NS
CHUNK = 96
N_CHUNKS = 36
N_PAIRS = N_CHUNKS // 2
TRIS_PER_W = CHUNK * N_CHUNKS
T_PAD = NW * TRIS_PER_W
WORDS = 2 * B
N_VERTS = 50000
VB = 64
N_FULL_BLKS = N_VERTS // VB
TAIL_V = N_VERTS - N_FULL_BLKS * VB
VW = 3 * VB
VROW = 193


def _sc_volume(x, mblk):
    mesh = plsc.VectorSubcoreMesh(core_axis_name="c", subcore_axis_name="s")

    @functools.partial(
        pl.kernel,
        mesh=mesh,
        out_type=jax.ShapeDtypeStruct((NW, B), jnp.float32),
        compiler_params=pltpu.CompilerParams(use_tc_tiling_on_sc=False,
                                             needs_layout_passes=False),
        scratch_types=[
            pltpu.VMEM((3, CHUNK), jnp.int32),
            pltpu.VMEM((3, CHUNK), jnp.int32),
            pltpu.VMEM((3, CHUNK, WORDS), jnp.int32),
            pltpu.VMEM((3, CHUNK, WORDS), jnp.int32),
            pltpu.VMEM((B, VROW), jnp.float32),
            pltpu.VMEM((B, VROW), jnp.float32),
            pltpu.VMEM((VB, WORDS), jnp.int32),
            pltpu.VMEM((VB, WORDS), jnp.int32),
            pltpu.VMEM((B,), jnp.float32),
            pltpu.VMEM_SHARED((N_VERTS, WORDS), jnp.int32),
            pltpu.SemaphoreType.DMA,
            pltpu.SemaphoreType.DMA,
            pltpu.SemaphoreType.DMA,
            pltpu.SemaphoreType.DMA,
            pltpu.SemaphoreType.DMA,
            pltpu.SemaphoreType.DMA,
        ],
    )
    def k(x_hbm, m_hbm, out_hbm,
          ia, ib, ra, rb, va, vb_, wa, wb, accv, tab,
          sem_a, sem_b, sem_va, sem_vb, sem_wa, sem_wb):
        cid = lax.axis_index("c")
        sid = lax.axis_index("s")
        wid = sid * NC + cid

        lane = lax.broadcasted_iota(jnp.int32, (B,), 0)
        zero16 = jnp.zeros((B,), jnp.float32)

        def stage(vbuf, sem, g):
            pltpu.async_copy(x_hbm.at[:, pl.ds(g * VW, VW)],
                             vbuf.at[:, pl.ds(0, VW)], sem)

        def stage_wait(vbuf, sem, g):
            pltpu.make_async_copy(x_hbm.at[:, pl.ds(g * VW, VW)],
                                  vbuf.at[:, pl.ds(0, VW)], sem).wait()

        def flush(wbuf, sem, g):
            pltpu.async_copy(wbuf, tab.at[pl.ds(g * VB, VB)], sem)

        def flush_wait(wbuf, sem, g):
            pltpu.make_async_copy(wbuf, tab.at[pl.ds(g * VB, VB)],
                                  sem).wait()

        def build(vbuf, wbuf, nv):
            def vert(dv, _):
                col = 3 * dv
                c0 = plsc.load_gather(vbuf, [lane, jnp.full((B,), col,
                                                            jnp.int32)])
                c1 = plsc.load_gather(vbuf, [lane, jnp.full((B,), col + 1,
                                                            jnp.int32)])
                c2 = plsc.load_gather(vbuf, [lane, jnp.full((B,), col + 2,
                                                            jnp.int32)])
                w01 = plsc.bitcast(
                    plsc.pack(c0, c1, format=plsc.PackFormat.INTERLEAVED),
                    jnp.int32)
                w2 = plsc.bitcast(
                    plsc.pack(c2, zero16,
                              format=plsc.PackFormat.INTERLEAVED),
                    jnp.int32)
                wbuf[dv, pl.ds(0, B)] = w01
                wbuf[dv, pl.ds(B, B)] = w2
                return 0

            lax.fori_loop(0, nv, vert, 0, unroll=4)

        n_sb = jnp.where(sid < N_FULL_BLKS - 48 * NS, 49, 48)
        g0 = sid

        stage(va, sem_va, g0)

        def build_pair(p, _):
            gA = g0 + 32 * p
            gB = gA + 16

            @pl.when(2 * p < n_sb)
            def _():
                @pl.when(2 * p + 1 < n_sb)
                def _():
                    stage(vb_, sem_vb, gB)

                @pl.when(p > 0)
                def _():
                    flush_wait(wa, sem_wa, gA)
                stage_wait(va, sem_va, gA)
                build(va, wa, VB)
                flush(wa, sem_wa, gA)

                @pl.when(2 * p + 2 < n_sb)
                def _():
                    stage(va, sem_va, gA + 32)

            @pl.when(2 * p + 1 < n_sb)
            def _():
                @pl.when(p > 0)
                def _():
                    flush_wait(wb, sem_wb, gB)
                stage_wait(vb_, sem_vb, gB)
                build(vb_, wb, VB)
                flush(wb, sem_wb, gB)

            return 0

        lax.fori_loop(0, 25, build_pair, 0)

        @pl.when(n_sb >= 1)
        def _():
            flush_wait(wa, sem_wa, 0)

        @pl.when(n_sb >= 2)
        def _():
            flush_wait(wb, sem_wb, 0)

        @pl.when(sid == 0)
        def _():
            pltpu.sync_copy(x_hbm.at[:, pl.ds(N_FULL_BLKS * VW, 3 * TAIL_V)],
                            va.at[:, pl.ds(0, 3 * TAIL_V)])
            build(va, wa, TAIL_V)
            pltpu.sync_copy(wa.at[pl.ds(0, TAIL_V)],
                            tab.at[pl.ds(N_FULL_BLKS * VB, TAIL_V)])

        plsc.subcore_barrier()

        def fetch(ibuf, rbuf, sem, ci):
            pltpu.sync_copy(m_hbm.at[wid * N_CHUNKS + ci], ibuf)
            for s in range(3):
                pltpu.async_copy(tab.at[ibuf.at[s]], rbuf.at[s], sem)

        def wait(ibuf, rbuf, sem):
            for s in range(3):
                pltpu.make_async_copy(tab.at[ibuf.at[s]],
                                      rbuf.at[s], sem).wait()

        def comp3(rbuf, s, t):
            w01 = rbuf[s, t, pl.ds(0, B)]
            c0, c1 = plsc.unpack(plsc.bitcast(w01, jnp.bfloat16),
                                 format=plsc.PackFormat.INTERLEAVED,
                                 preferred_element_type=jnp.float32)
            w2 = rbuf[s, t, pl.ds(B, B)]
            c2, _ = plsc.unpack(plsc.bitcast(w2, jnp.bfloat16),
                                format=plsc.PackFormat.INTERLEAVED,
                                preferred_element_type=jnp.float32)
            return c0, c1, c2

        def compute(rbuf, acc):
            def tri_body(t, a):
                a1, a2, a3 = comp3(rbuf, 0, t)
                b1, b2, b3 = comp3(rbuf, 1, t)
                d1, d2, d3 = comp3(rbuf, 2, t)
                det = (a1 * (b2 * d3 - b3 * d2)
                       - a2 * (b1 * d3 - b3 * d1)
                       + a3 * (b1 * d2 - b2 * d1))
                return a + jnp.abs(det)

            return lax.fori_loop(0, CHUNK, tri_body, acc, unroll=4)

        fetch(ia, ra, sem_a, 0)

        def pair_body(p, acc):
            c0 = 2 * p
            fetch(ib, rb, sem_b, c0 + 1)
            wait(ia, ra, sem_a)
            acc = compute(ra, acc)

            @pl.when(p < N_PAIRS - 1)
            def _():
                fetch(ia, ra, sem_a, c0 + 2)

            wait(ib, rb, sem_b)
            return compute(rb, acc)

        acc = lax.fori_loop(0, N_PAIRS, pair_body,
                            jnp.zeros((B,), jnp.float32))
        accv[...] = acc
        pltpu.sync_copy(accv, out_hbm.at[wid])

    return k(x, mblk)


_XBLK = 12288


def _normalize(x, partials):
    cols = x.shape[1]
    grid = pl.cdiv(cols, _XBLK)

    def body(p_ref, x_ref, o_ref):
        tot = jnp.sum(p_ref[...], axis=0)
        vol = tot * (1.0 / 6.0)
        inv = jnp.exp(jnp.log(vol) * (-1.0 / 3.0))
        o_ref[...] = x_ref[...] * inv[:, None]

    return pl.pallas_call(
        body,
        grid=(grid,),
        in_specs=[
            pl.BlockSpec((NW, B), lambda i: (0, 0)),
            pl.BlockSpec((B, _XBLK), lambda i: (0, i)),
        ],
        out_specs=pl.BlockSpec((B, _XBLK), lambda i: (0, i)),
        out_shape=jax.ShapeDtypeStruct(x.shape, x.dtype),
    )(partials, x)


def kernel(x, M):
    Mi = M.astype(jnp.int32)
    pad = T_PAD - Mi.shape[0]
    Mp = jnp.concatenate([Mi, jnp.zeros((pad, 3), jnp.int32)], axis=0)
    mblk = Mp.reshape(NW * N_CHUNKS, CHUNK, 3).transpose(0, 2, 1)
    partials = _sc_volume(x, mblk)
    return _normalize(x, partials)
